# Initial kernel scaffold; baseline (speedup 1.0000x reference)
import jax
import jax.numpy as jnp
from jax.experimental import pallas as pl


def kernel(Gu, Gi, W, b, user, item, neigh_user, neigh_item):
    raise NotImplementedError("write your pallas kernel here")


# speedup vs baseline: 4.5365x; 4.5365x over previous
import functools

import jax
import jax.numpy as jnp
from jax import lax
from jax.experimental import pallas as pl
from jax.experimental.pallas import tpu as pltpu
from jax.experimental.pallas import tpu_sc as plsc

EMBED_K = 64
DISEN_K = 4
D_K = EMBED_K // DISEN_K
TAU = 0.1
ROUTING_ITERS = 3
N_NEIGH = 4096
EPS = 1e-12

try:
    _info = plsc.get_sparse_core_info()
    _NC = _info.num_cores
    _NS = _info.num_subcores
except Exception:
    _NC, _NS = 2, 16
_NW = _NC * _NS
_BPW = N_NEIGH // _NW


def _sc_gather(Gu, Gi, idx_nu, idx_ni, idx_cu, idx_ci):
    f32 = jnp.float32
    mesh = plsc.VectorSubcoreMesh(core_axis_name="c", subcore_axis_name="s")

    @functools.partial(
        pl.kernel,
        mesh=mesh,
        compiler_params=pltpu.CompilerParams(use_tc_tiling_on_sc=False),
        out_type=[
            jax.ShapeDtypeStruct((N_NEIGH, EMBED_K), f32),
            jax.ShapeDtypeStruct((N_NEIGH, EMBED_K), f32),
            jax.ShapeDtypeStruct((8, EMBED_K), f32),
            jax.ShapeDtypeStruct((8, EMBED_K), f32),
        ],
        scratch_types=[
            pltpu.VMEM((_BPW,), jnp.int32),
            pltpu.VMEM((_BPW, EMBED_K), f32),
            pltpu.VMEM((_BPW,), jnp.int32),
            pltpu.VMEM((_BPW, EMBED_K), f32),
            pltpu.VMEM((8,), jnp.int32),
            pltpu.VMEM((8, EMBED_K), f32),
            pltpu.SemaphoreType.DMA,
            pltpu.SemaphoreType.DMA,
        ],
    )
    def gk(gu_h, gi_h, inu_h, ini_h, icu_h, ici_h, onu, oni, ocu, oci,
           idx_a, rows_a, idx_b, rows_b, cidx, crows, sem_a, sem_b):
        wid = lax.axis_index("s") * _NC + lax.axis_index("c")
        base = wid * _BPW
        pltpu.sync_copy(inu_h.at[pl.ds(base, _BPW)], idx_a)
        pltpu.sync_copy(ini_h.at[pl.ds(base, _BPW)], idx_b)
        ca = pltpu.async_copy(gi_h.at[idx_a], rows_a, sem_a)
        cb = pltpu.async_copy(gu_h.at[idx_b], rows_b, sem_b)
        ca.wait()
        pltpu.sync_copy(rows_a, onu.at[pl.ds(base, _BPW)])
        cb.wait()
        pltpu.sync_copy(rows_b, oni.at[pl.ds(base, _BPW)])

        @pl.when(wid == 0)
        def _():
            pltpu.sync_copy(icu_h, cidx)
            pltpu.async_copy(gu_h.at[cidx], crows, sem_a).wait()
            pltpu.sync_copy(crows, ocu)
            pltpu.sync_copy(ici_h, cidx)
            pltpu.async_copy(gi_h.at[cidx], crows, sem_b).wait()
            pltpu.sync_copy(crows, oci)

    return gk(Gu, Gi, idx_nu, idx_ni, idx_cu, idx_ci)


def _tc_body(nbr_u_ref, nbr_i_ref, cu_ref, ci_ref, wr_ref, br_ref, out_ref):
    f32 = jnp.float32
    wr = wr_ref[...]
    br = br_ref[0:1, :]
    rr = lax.broadcasted_iota(jnp.int32, (EMBED_K, EMBED_K), 0) // D_K
    cc = lax.broadcasted_iota(jnp.int32, (EMBED_K, EMBED_K), 1) // D_K
    s_mat = (rr == cc).astype(f32)

    def project(x):
        h = jnp.maximum(jnp.dot(x, wr, preferred_element_type=f32) + br, 0.0)
        ss = jnp.dot(h * h, s_mat, preferred_element_type=f32)
        return h / (jnp.sqrt(ss) + EPS)

    z_u = project(nbr_u_ref[...])
    z_i = project(nbr_i_ref[...])
    e_u = project(cu_ref[...])[0:1, :]
    e_i = project(ci_ref[...])[0:1, :]

    def routing(z, e):
        logits = jnp.dot(z * e, s_mat, preferred_element_type=f32) * (1.0 / TAU)
        m = jnp.max(logits, axis=1, keepdims=True)
        ex = jnp.exp(logits - m)
        denom = jnp.sum(ex, axis=1, keepdims=True) * (1.0 / D_K)
        p = ex / denom
        agg = jnp.sum(p * z, axis=0, keepdims=True)
        v = e + agg
        ss = jnp.dot(v * v, s_mat, preferred_element_type=f32)
        return v / (jnp.sqrt(ss) + EPS)

    for _ in range(ROUTING_ITERS):
        e_u = routing(z_u, e_u)
        e_i = routing(z_i, e_i)

    xui = jnp.sum(e_u * e_i)
    out_ref[0:1, :] = e_u
    out_ref[1:2, :] = e_i
    out_ref[2:3, :] = jnp.zeros((1, EMBED_K), f32) + xui
    out_ref[3:8, :] = jnp.zeros((5, EMBED_K), f32)


def kernel(Gu, Gi, W, b, user, item, neigh_user, neigh_item):
    idx_cu = jnp.broadcast_to(user, (8,)).astype(jnp.int32)
    idx_ci = jnp.broadcast_to(item, (8,)).astype(jnp.int32)
    nbr_u, nbr_i, cu, ci = _sc_gather(Gu, Gi, neigh_user, neigh_item,
                                      idx_cu, idx_ci)
    wr = jnp.transpose(W, (1, 0, 2)).reshape(EMBED_K, DISEN_K * D_K)
    br = jnp.broadcast_to(b.reshape(1, DISEN_K * D_K), (8, DISEN_K * D_K))
    out = pl.pallas_call(
        _tc_body,
        out_shape=jax.ShapeDtypeStruct((8, EMBED_K), jnp.float32),
    )(nbr_u, nbr_i, cu, ci, wr, br)
    return (out[2, 0:1], out[0], out[1])


# speedup vs baseline: 7.1372x; 1.5733x over previous
import functools

import jax
import jax.numpy as jnp
from jax import lax
from jax.experimental import pallas as pl
from jax.experimental.pallas import tpu as pltpu
from jax.experimental.pallas import tpu_sc as plsc

EMBED_K = 64
DISEN_K = 4
D_K = EMBED_K // DISEN_K
TAU = 0.1
ROUTING_ITERS = 3
N_NEIGH = 4096
EPS = 1e-12

try:
    _info = plsc.get_sparse_core_info()
    _NC = _info.num_cores
    _NS = _info.num_subcores
except Exception:
    _NC, _NS = 2, 16
_NW = _NC * _NS
_BPW = N_NEIGH // _NW


def _sc_gather(Gu, Gi, idx_nu, idx_ni, idx_cu, idx_ci):
    f32 = jnp.float32
    mesh = plsc.VectorSubcoreMesh(core_axis_name="c", subcore_axis_name="s")

    @functools.partial(
        pl.kernel,
        mesh=mesh,
        out_type=[
            jax.ShapeDtypeStruct((N_NEIGH, EMBED_K), f32),
            jax.ShapeDtypeStruct((N_NEIGH, EMBED_K), f32),
            jax.ShapeDtypeStruct((8, EMBED_K), f32),
            jax.ShapeDtypeStruct((8, EMBED_K), f32),
        ],
        scratch_types=[
            pltpu.VMEM((_BPW + 16, EMBED_K), f32),
            pltpu.VMEM((_BPW + 16, EMBED_K), f32),
            pltpu.VMEM((8, EMBED_K), f32),
            pltpu.VMEM((_BPW + 16,), jnp.int32),
            pltpu.VMEM((_BPW + 16,), jnp.int32),
            pltpu.VMEM((16,), jnp.int32),
            pltpu.SemaphoreType.DMA,
            pltpu.SemaphoreType.DMA,
        ],
    )
    def gk(gu_h, gi_h, inu_h, ini_h, icu_h, ici_h, onu, oni, ocu, oci,
           rows_a, rows_b, crows, idx_a, idx_b, cidx, sem_a, sem_b):
        wid = lax.axis_index("s") * _NC + lax.axis_index("c")
        base = wid * _BPW
        pltpu.sync_copy(inu_h.at[pl.ds(base, _BPW)], idx_a.at[pl.ds(0, _BPW)])
        pltpu.sync_copy(ini_h.at[pl.ds(base, _BPW)], idx_b.at[pl.ds(0, _BPW)])

        def fire(i, _):
            ia = idx_a[pl.ds(i, 16)][0]
            pltpu.make_async_copy(
                gi_h.at[pl.ds(ia, 1)], rows_a.at[pl.ds(i, 1)], sem_a).start()
            ib = idx_b[pl.ds(i, 16)][0]
            pltpu.make_async_copy(
                gu_h.at[pl.ds(ib, 1)], rows_b.at[pl.ds(i, 1)], sem_b).start()
            return _

        lax.fori_loop(0, _BPW, fire, 0)
        pltpu.make_async_copy(
            gi_h.at[pl.ds(0, _BPW)], rows_a.at[pl.ds(0, _BPW)], sem_a).wait()
        pltpu.make_async_copy(
            gu_h.at[pl.ds(0, _BPW)], rows_b.at[pl.ds(0, _BPW)], sem_b).wait()
        pltpu.sync_copy(rows_a.at[pl.ds(0, _BPW)], onu.at[pl.ds(base, _BPW)])
        pltpu.sync_copy(rows_b.at[pl.ds(0, _BPW)], oni.at[pl.ds(base, _BPW)])

        @pl.when(wid == 0)
        def _():
            pltpu.sync_copy(icu_h.at[pl.ds(0, 16)], cidx)
            iu = cidx[pl.ds(0, 16)][0]
            pltpu.make_async_copy(
                gu_h.at[pl.ds(iu, 1)], crows.at[pl.ds(0, 1)], sem_a).start()
            pltpu.sync_copy(ici_h.at[pl.ds(0, 16)], cidx)
            ii = cidx[pl.ds(0, 16)][0]
            pltpu.make_async_copy(
                gi_h.at[pl.ds(ii, 1)], crows.at[pl.ds(1, 1)], sem_a).start()
            pltpu.make_async_copy(
                gu_h.at[pl.ds(0, 2)], crows.at[pl.ds(0, 2)], sem_a).wait()
            pltpu.sync_copy(crows.at[pl.ds(0, 1)], ocu.at[pl.ds(0, 1)])
            pltpu.sync_copy(crows.at[pl.ds(1, 1)], oci.at[pl.ds(0, 1)])

    return gk(Gu, Gi, idx_nu, idx_ni, idx_cu, idx_ci)


def _tc_body(nbr_u_ref, nbr_i_ref, cu_ref, ci_ref, wr_ref, br_ref, out_ref):
    f32 = jnp.float32
    wr = wr_ref[...]
    br = br_ref[0:1, :]
    rr = lax.broadcasted_iota(jnp.int32, (EMBED_K, EMBED_K), 0) // D_K
    cc = lax.broadcasted_iota(jnp.int32, (EMBED_K, EMBED_K), 1) // D_K
    s_mat = (rr == cc).astype(f32)

    def project(x):
        h = jnp.maximum(jnp.dot(x, wr, preferred_element_type=f32) + br, 0.0)
        ss = jnp.dot(h * h, s_mat, preferred_element_type=f32)
        return h / (jnp.sqrt(ss) + EPS)

    z_u = project(nbr_u_ref[...])
    z_i = project(nbr_i_ref[...])
    e_u = project(cu_ref[...])[0:1, :]
    e_i = project(ci_ref[...])[0:1, :]

    def routing(z, e):
        logits = jnp.dot(z * e, s_mat, preferred_element_type=f32) * (1.0 / TAU)
        m = jnp.max(logits, axis=1, keepdims=True)
        ex = jnp.exp(logits - m)
        denom = jnp.sum(ex, axis=1, keepdims=True) * (1.0 / D_K)
        p = ex / denom
        agg = jnp.sum(p * z, axis=0, keepdims=True)
        v = e + agg
        ss = jnp.dot(v * v, s_mat, preferred_element_type=f32)
        return v / (jnp.sqrt(ss) + EPS)

    for _ in range(ROUTING_ITERS):
        e_u = routing(z_u, e_u)
        e_i = routing(z_i, e_i)

    xui = jnp.sum(e_u * e_i)
    out_ref[0:1, :] = e_u
    out_ref[1:2, :] = e_i
    out_ref[2:3, :] = jnp.zeros((1, EMBED_K), f32) + xui
    out_ref[3:8, :] = jnp.zeros((5, EMBED_K), f32)


def kernel(Gu, Gi, W, b, user, item, neigh_user, neigh_item):
    idx_cu = jnp.broadcast_to(user, (16,)).astype(jnp.int32)
    idx_ci = jnp.broadcast_to(item, (16,)).astype(jnp.int32)
    nbr_u, nbr_i, cu, ci = _sc_gather(Gu, Gi, neigh_user, neigh_item,
                                      idx_cu, idx_ci)
    wr = jnp.transpose(W, (1, 0, 2)).reshape(EMBED_K, DISEN_K * D_K)
    br = jnp.broadcast_to(b.reshape(1, DISEN_K * D_K), (8, DISEN_K * D_K))
    out = pl.pallas_call(
        _tc_body,
        out_shape=jax.ShapeDtypeStruct((8, EMBED_K), jnp.float32),
    )(nbr_u, nbr_i, cu, ci, wr, br)
    return (out[2, 0:1], out[0], out[1])


# speedup vs baseline: 29.6173x; 4.1497x over previous
import functools

import jax
import jax.numpy as jnp
from jax import lax
from jax.experimental import pallas as pl
from jax.experimental.pallas import tpu as pltpu
from jax.experimental.pallas import tpu_sc as plsc

EMBED_K = 64
DISEN_K = 4
D_K = EMBED_K // DISEN_K
TAU = 0.1
ROUTING_ITERS = 3
N_NEIGH = 4096
EPS = 1e-12

try:
    _info = plsc.get_sparse_core_info()
    _NC = _info.num_cores
    _NS = _info.num_subcores
except Exception:
    _NC, _NS = 2, 16
_NW = _NC * _NS
_BPW = N_NEIGH // _NW
_K = 4


def _sc_gather(gut, git, idx_nu, idx_ni, idx_cu, idx_ci):
    f32 = jnp.float32
    mesh = plsc.VectorSubcoreMesh(core_axis_name="c", subcore_axis_name="s")

    @functools.partial(
        pl.kernel,
        mesh=mesh,
        compiler_params=pltpu.CompilerParams(needs_layout_passes=False),
        out_type=[
            jax.ShapeDtypeStruct((EMBED_K, N_NEIGH), f32),
            jax.ShapeDtypeStruct((EMBED_K, N_NEIGH), f32),
            jax.ShapeDtypeStruct((EMBED_K, 8), f32),
        ],
        scratch_types=[
            pltpu.VMEM((2 * _K, EMBED_K, 128), f32),
            pltpu.VMEM((EMBED_K, _BPW), f32),
            pltpu.VMEM((EMBED_K, _BPW), f32),
            pltpu.VMEM((EMBED_K, 8), f32),
            pltpu.VMEM((_BPW + 16,), jnp.int32),
            pltpu.VMEM((_BPW + 16,), jnp.int32),
            pltpu.VMEM((16,), jnp.int32),
            pltpu.SemaphoreType.DMA,
            pltpu.SemaphoreType.DMA,
        ],
    )
    def gk(gut_h, git_h, inu_h, ini_h, icu_h, ici_h, oxu, oxi, oc,
           blks, cols_a, cols_b, ccols, idx_a, idx_b, cidx, sem_a, sem_b):
        wid = lax.axis_index("s") * _NC + lax.axis_index("c")
        base = wid * _BPW
        pltpu.sync_copy(inu_h.at[pl.ds(base, _BPW)], idx_a.at[pl.ds(0, _BPW)])
        pltpu.sync_copy(ini_h.at[pl.ds(base, _BPW)], idx_b.at[pl.ds(0, _BPW)])

        def scalar_at(idx_ref, i):
            return idx_ref[pl.ds(i, 16)][0]

        def fetch_block(tbl, ref_idx, slot, sem):
            blk = pl.multiple_of((ref_idx // 128) * 128, 128)
            pltpu.make_async_copy(
                tbl.at[:, pl.ds(blk, 128)], blks.at[slot], sem).start()

        def extract(ref_idx, slot, cols, i):
            lane = lax.rem(ref_idx, 128)
            lanev = jnp.zeros((16,), jnp.int32) + lane
            slotv = jnp.zeros((16,), jnp.int32) + slot
            iv = jnp.zeros((16,), jnp.int32) + i
            for g in range(EMBED_K // 16):
                rows = lax.iota(jnp.int32, 16) + (16 * g)
                v = plsc.load_gather(blks, [slotv, rows, lanev])
                plsc.store_scatter(cols, [rows, iv], v)

        def chunk(c, _):
            i0 = c * _K
            for s in range(_K):
                fetch_block(git_h, scalar_at(idx_a, i0 + s), s, sem_a)
                fetch_block(gut_h, scalar_at(idx_b, i0 + s), _K + s, sem_b)
            for s in range(_K):
                pltpu.make_async_copy(
                    git_h.at[:, pl.ds(0, 128)], blks.at[s], sem_a).wait()
                pltpu.make_async_copy(
                    gut_h.at[:, pl.ds(0, 128)], blks.at[_K + s], sem_b).wait()
            for s in range(_K):
                extract(scalar_at(idx_a, i0 + s), s, cols_a, i0 + s)
                extract(scalar_at(idx_b, i0 + s), _K + s, cols_b, i0 + s)
            return _

        lax.fori_loop(0, _BPW // _K, chunk, 0)
        obase = pl.multiple_of(base, 128)
        pltpu.sync_copy(cols_a, oxu.at[:, pl.ds(obase, _BPW)])
        pltpu.sync_copy(cols_b, oxi.at[:, pl.ds(obase, _BPW)])

        @pl.when(wid == 0)
        def _():
            pltpu.sync_copy(icu_h.at[pl.ds(0, 16)], cidx)
            iu = scalar_at(cidx, 0)
            fetch_block(gut_h, iu, 0, sem_a)
            pltpu.make_async_copy(
                gut_h.at[:, pl.ds(0, 128)], blks.at[0], sem_a).wait()
            extract(iu, 0, ccols, 0)
            pltpu.sync_copy(ici_h.at[pl.ds(0, 16)], cidx)
            ii = scalar_at(cidx, 0)
            fetch_block(git_h, ii, 0, sem_a)
            pltpu.make_async_copy(
                git_h.at[:, pl.ds(0, 128)], blks.at[0], sem_a).wait()
            extract(ii, 0, ccols, 1)
            pltpu.sync_copy(ccols, oc)

    return gk(gut, git, idx_nu, idx_ni, idx_cu, idx_ci)


def _tc_body(xtu_ref, xti_ref, c_ref, wrt_ref, bc_ref, out_ref):
    f32 = jnp.float32
    wrt = wrt_ref[...]
    bc = bc_ref[:, 0:1]
    rr = lax.broadcasted_iota(jnp.int32, (EMBED_K, EMBED_K), 0) // D_K
    cc = lax.broadcasted_iota(jnp.int32, (EMBED_K, EMBED_K), 1) // D_K
    s_mat = (rr == cc).astype(f32)

    def project(xt):
        h = jnp.maximum(
            jnp.dot(wrt, xt, preferred_element_type=f32) + bc, 0.0)
        ss = jnp.dot(s_mat, h * h, preferred_element_type=f32)
        return h / (jnp.sqrt(ss) + EPS)

    zt_u = project(xtu_ref[...])
    zt_i = project(xti_ref[...])
    zc = project(c_ref[...])
    e_u = zc[:, 0:1]
    e_i = zc[:, 1:2]

    def routing(zt, e):
        logits = jnp.dot(s_mat, zt * e, preferred_element_type=f32) * (1.0 / TAU)
        m = jnp.max(logits, axis=0, keepdims=True)
        ex = jnp.exp(logits - m)
        denom = jnp.sum(ex, axis=0, keepdims=True) * (1.0 / D_K)
        p = ex / denom
        agg = jnp.sum(p * zt, axis=1, keepdims=True)
        v = e + agg
        ss = jnp.dot(s_mat, v * v, preferred_element_type=f32)
        return v / (jnp.sqrt(ss) + EPS)

    for _ in range(ROUTING_ITERS):
        e_u = routing(zt_u, e_u)
        e_i = routing(zt_i, e_i)

    xui = jnp.sum(e_u * e_i)
    out_ref[:, 0:1] = e_u
    out_ref[:, 1:2] = e_i
    out_ref[:, 2:3] = jnp.zeros((EMBED_K, 1), f32) + xui
    out_ref[:, 3:8] = jnp.zeros((EMBED_K, 5), f32)


def kernel(Gu, Gi, W, b, user, item, neigh_user, neigh_item):
    gut = Gu.T
    git = Gi.T
    idx_cu = jnp.broadcast_to(user, (16,)).astype(jnp.int32)
    idx_ci = jnp.broadcast_to(item, (16,)).astype(jnp.int32)
    xtu, xti, c2 = _sc_gather(gut, git, neigh_user, neigh_item,
                              idx_cu, idx_ci)
    wrt = jnp.transpose(W, (0, 2, 1)).reshape(DISEN_K * D_K, EMBED_K)
    bc = jnp.broadcast_to(b.reshape(DISEN_K * D_K, 1), (DISEN_K * D_K, 8))
    out = pl.pallas_call(
        _tc_body,
        out_shape=jax.ShapeDtypeStruct((EMBED_K, 8), jnp.float32),
    )(xtu, xti, c2, wrt, bc)
    return (out[0:1, 2].reshape(1), out[:, 0], out[:, 1])


# speedup vs baseline: 38.8782x; 1.3127x over previous
import functools

import jax
import jax.numpy as jnp
from jax import lax
from jax.experimental import pallas as pl
from jax.experimental.pallas import tpu as pltpu
from jax.experimental.pallas import tpu_sc as plsc

EMBED_K = 64
DISEN_K = 4
D_K = EMBED_K // DISEN_K
TAU = 0.1
ROUTING_ITERS = 3
N_NEIGH = 4096
EPS = 1e-12

try:
    _info = plsc.get_sparse_core_info()
    _NC = _info.num_cores
    _NS = _info.num_subcores
except Exception:
    _NC, _NS = 2, 16
_NW = _NC * _NS
_BPW = N_NEIGH // _NW
_K = 4


def _sc_gather(gut, git, idx_nu, idx_ni, idx_cu, idx_ci):
    f32 = jnp.float32
    mesh = plsc.VectorSubcoreMesh(core_axis_name="c", subcore_axis_name="s")

    @functools.partial(
        pl.kernel,
        mesh=mesh,
        compiler_params=pltpu.CompilerParams(needs_layout_passes=False),
        out_type=[
            jax.ShapeDtypeStruct((EMBED_K, N_NEIGH), f32),
            jax.ShapeDtypeStruct((EMBED_K, N_NEIGH), f32),
            jax.ShapeDtypeStruct((EMBED_K, 8), f32),
        ],
        scratch_types=[
            pltpu.VMEM((2 * _K, EMBED_K, 128), f32),
            pltpu.VMEM((EMBED_K, _BPW), f32),
            pltpu.VMEM((EMBED_K, _BPW), f32),
            pltpu.VMEM((EMBED_K, 8), f32),
            pltpu.VMEM((_BPW + 16,), jnp.int32),
            pltpu.VMEM((_BPW + 16,), jnp.int32),
            pltpu.VMEM((16,), jnp.int32),
            pltpu.SemaphoreType.DMA,
            pltpu.SemaphoreType.DMA,
        ],
    )
    def gk(gut_h, git_h, inu_h, ini_h, icu_h, ici_h, oxu, oxi, oc,
           blks, cols_a, cols_b, ccols, idx_a, idx_b, cidx, sem_a, sem_b):
        wid = lax.axis_index("s") * _NC + lax.axis_index("c")
        base = wid * _BPW
        pltpu.sync_copy(inu_h.at[pl.ds(base, _BPW)], idx_a.at[pl.ds(0, _BPW)])
        pltpu.sync_copy(ini_h.at[pl.ds(base, _BPW)], idx_b.at[pl.ds(0, _BPW)])

        def scalar_at(idx_ref, i):
            return idx_ref[pl.ds(i, 16)][0]

        def fetch_block(tbl, ref_idx, slot, sem):
            blk = pl.multiple_of((ref_idx // 128) * 128, 128)
            pltpu.make_async_copy(
                tbl.at[:, pl.ds(blk, 128)], blks.at[slot], sem).start()

        def extract(ref_idx, slot, cols, i):
            lane = lax.rem(ref_idx, 128)
            lanev = jnp.zeros((16,), jnp.int32) + lane
            slotv = jnp.zeros((16,), jnp.int32) + slot
            iv = jnp.zeros((16,), jnp.int32) + i
            for g in range(EMBED_K // 16):
                rows = lax.iota(jnp.int32, 16) + (16 * g)
                v = plsc.load_gather(blks, [slotv, rows, lanev])
                plsc.store_scatter(cols, [rows, iv], v)

        def fire(i, s):
            fetch_block(git_h, scalar_at(idx_a, i), s, sem_a)
            fetch_block(gut_h, scalar_at(idx_b, i), _K + s, sem_b)

        for s in range(_K):
            fire(s, s)

        def group(g, carry):
            for s in range(_K):
                i = g * _K + s
                pltpu.make_async_copy(
                    git_h.at[:, pl.ds(0, 128)], blks.at[s], sem_a).wait()
                pltpu.make_async_copy(
                    gut_h.at[:, pl.ds(0, 128)], blks.at[_K + s], sem_b).wait()
                extract(scalar_at(idx_a, i), s, cols_a, i)
                extract(scalar_at(idx_b, i), _K + s, cols_b, i)

                @pl.when(i + _K < _BPW)
                def _refire():
                    fire(i + _K, s)
            return carry

        lax.fori_loop(0, _BPW // _K, group, 0)
        obase = pl.multiple_of(base, 128)
        pltpu.sync_copy(cols_a, oxu.at[:, pl.ds(obase, _BPW)])
        pltpu.sync_copy(cols_b, oxi.at[:, pl.ds(obase, _BPW)])

        @pl.when(wid == 0)
        def _():
            pltpu.sync_copy(icu_h.at[pl.ds(0, 16)], cidx)
            iu = scalar_at(cidx, 0)
            fetch_block(gut_h, iu, 0, sem_a)
            pltpu.make_async_copy(
                gut_h.at[:, pl.ds(0, 128)], blks.at[0], sem_a).wait()
            extract(iu, 0, ccols, 0)
            pltpu.sync_copy(ici_h.at[pl.ds(0, 16)], cidx)
            ii = scalar_at(cidx, 0)
            fetch_block(git_h, ii, 0, sem_a)
            pltpu.make_async_copy(
                git_h.at[:, pl.ds(0, 128)], blks.at[0], sem_a).wait()
            extract(ii, 0, ccols, 1)
            pltpu.sync_copy(ccols, oc)

    return gk(gut, git, idx_nu, idx_ni, idx_cu, idx_ci)


def _tc_body(xtu_ref, xti_ref, c_ref, wrt_ref, bc_ref, out_ref):
    f32 = jnp.float32
    wrt = wrt_ref[...]
    bc = bc_ref[:, 0:1]
    rr = lax.broadcasted_iota(jnp.int32, (EMBED_K, EMBED_K), 0) // D_K
    cc = lax.broadcasted_iota(jnp.int32, (EMBED_K, EMBED_K), 1) // D_K
    s_mat = (rr == cc).astype(f32)

    def project(xt):
        h = jnp.maximum(
            jnp.dot(wrt, xt, preferred_element_type=f32) + bc, 0.0)
        ss = jnp.dot(s_mat, h * h, preferred_element_type=f32)
        return h / (jnp.sqrt(ss) + EPS)

    zt_u = project(xtu_ref[...])
    zt_i = project(xti_ref[...])
    zc = project(c_ref[...])
    e_u = zc[:, 0:1]
    e_i = zc[:, 1:2]

    def routing(zt, e):
        logits = jnp.dot(s_mat, zt * e, preferred_element_type=f32) * (1.0 / TAU)
        m = jnp.max(logits, axis=0, keepdims=True)
        ex = jnp.exp(logits - m)
        denom = jnp.sum(ex, axis=0, keepdims=True) * (1.0 / D_K)
        p = ex / denom
        agg = jnp.sum(p * zt, axis=1, keepdims=True)
        v = e + agg
        ss = jnp.dot(s_mat, v * v, preferred_element_type=f32)
        return v / (jnp.sqrt(ss) + EPS)

    for _ in range(ROUTING_ITERS):
        e_u = routing(zt_u, e_u)
        e_i = routing(zt_i, e_i)

    xui = jnp.sum(e_u * e_i)
    out_ref[:, 0:1] = e_u
    out_ref[:, 1:2] = e_i
    out_ref[:, 2:3] = jnp.zeros((EMBED_K, 1), f32) + xui
    out_ref[:, 3:8] = jnp.zeros((EMBED_K, 5), f32)


def kernel(Gu, Gi, W, b, user, item, neigh_user, neigh_item):
    gut = Gu.T
    git = Gi.T
    idx_cu = jnp.broadcast_to(user, (16,)).astype(jnp.int32)
    idx_ci = jnp.broadcast_to(item, (16,)).astype(jnp.int32)
    xtu, xti, c2 = _sc_gather(gut, git, neigh_user, neigh_item,
                              idx_cu, idx_ci)
    wrt = jnp.transpose(W, (0, 2, 1)).reshape(DISEN_K * D_K, EMBED_K)
    bc = jnp.broadcast_to(b.reshape(DISEN_K * D_K, 1), (DISEN_K * D_K, 8))
    out = pl.pallas_call(
        _tc_body,
        out_shape=jax.ShapeDtypeStruct((EMBED_K, 8), jnp.float32),
    )(xtu, xti, c2, wrt, bc)
    return (out[0:1, 2].reshape(1), out[:, 0], out[:, 1])


# speedup vs baseline: 39.3104x; 1.0111x over previous
import functools

import jax
import jax.numpy as jnp
from jax import lax
from jax.experimental import pallas as pl
from jax.experimental.pallas import tpu as pltpu
from jax.experimental.pallas import tpu_sc as plsc

EMBED_K = 64
DISEN_K = 4
D_K = EMBED_K // DISEN_K
TAU = 0.1
ROUTING_ITERS = 3
N_NEIGH = 4096
EPS = 1e-12

try:
    _info = plsc.get_sparse_core_info()
    _NC = _info.num_cores
    _NS = _info.num_subcores
except Exception:
    _NC, _NS = 2, 16
_NW = _NC * _NS
_BPW = N_NEIGH // _NW
_K = 4


def _sc_gather(gut, git, idx_nu, idx_ni, idx_cu, idx_ci):
    f32 = jnp.float32
    mesh = plsc.VectorSubcoreMesh(core_axis_name="c", subcore_axis_name="s")

    @functools.partial(
        pl.kernel,
        mesh=mesh,
        compiler_params=pltpu.CompilerParams(needs_layout_passes=False),
        out_type=[
            jax.ShapeDtypeStruct((EMBED_K, N_NEIGH), f32),
            jax.ShapeDtypeStruct((EMBED_K, N_NEIGH), f32),
            jax.ShapeDtypeStruct((EMBED_K, 8), f32),
        ],
        scratch_types=[
            pltpu.VMEM((2 * _K, EMBED_K, 128), f32),
            pltpu.VMEM((EMBED_K, _BPW), f32),
            pltpu.VMEM((EMBED_K, _BPW), f32),
            pltpu.VMEM((EMBED_K, 8), f32),
            pltpu.VMEM((_BPW + 16,), jnp.int32),
            pltpu.VMEM((_BPW + 16,), jnp.int32),
            pltpu.VMEM((16,), jnp.int32),
            pltpu.SemaphoreType.DMA,
            pltpu.SemaphoreType.DMA,
        ],
    )
    def gk(gut_h, git_h, inu_h, ini_h, icu_h, ici_h, oxu, oxi, oc,
           blks, cols_a, cols_b, ccols, idx_a, idx_b, cidx, sem_a, sem_b):
        wid = lax.axis_index("s") * _NC + lax.axis_index("c")
        base = wid * _BPW
        pltpu.sync_copy(inu_h.at[pl.ds(base, _BPW)], idx_a.at[pl.ds(0, _BPW)])
        pltpu.sync_copy(ini_h.at[pl.ds(base, _BPW)], idx_b.at[pl.ds(0, _BPW)])

        def scalar_at(idx_ref, i):
            return idx_ref[pl.ds(i, 16)][0]

        def fetch_block(tbl, ref_idx, slot, sem):
            blk = pl.multiple_of((ref_idx // 128) * 128, 128)
            pltpu.make_async_copy(
                tbl.at[:, pl.ds(blk, 128)], blks.at[slot], sem).start()

        def extract(ref_idx, slot, cols, i):
            lane = lax.rem(ref_idx, 128)
            lanev = jnp.zeros((16,), jnp.int32) + lane
            slotv = jnp.zeros((16,), jnp.int32) + slot
            iv = jnp.zeros((16,), jnp.int32) + i
            for g in range(EMBED_K // 16):
                rows = lax.iota(jnp.int32, 16) + (16 * g)
                v = plsc.load_gather(blks, [slotv, rows, lanev])
                plsc.store_scatter(cols, [rows, iv], v)

        def fire(i, s):
            fetch_block(git_h, scalar_at(idx_a, i), s, sem_a)
            fetch_block(gut_h, scalar_at(idx_b, i), _K + s, sem_b)

        for s in range(_K):
            fire(s, s)

        def group(g, carry):
            for s in range(_K):
                i = g * _K + s
                pltpu.make_async_copy(
                    git_h.at[:, pl.ds(0, 128)], blks.at[s], sem_a).wait()
                extract(scalar_at(idx_a, i), s, cols_a, i)
                pltpu.make_async_copy(
                    gut_h.at[:, pl.ds(0, 128)], blks.at[_K + s], sem_b).wait()
                extract(scalar_at(idx_b, i), _K + s, cols_b, i)

                @pl.when(i + _K < _BPW)
                def _refire():
                    fire(i + _K, s)
            return carry

        lax.fori_loop(0, _BPW // _K, group, 0)
        obase = pl.multiple_of(base, 128)
        pltpu.sync_copy(cols_a, oxu.at[:, pl.ds(obase, _BPW)])
        pltpu.sync_copy(cols_b, oxi.at[:, pl.ds(obase, _BPW)])

        @pl.when(wid == 0)
        def _():
            pltpu.sync_copy(icu_h.at[pl.ds(0, 16)], cidx)
            iu = scalar_at(cidx, 0)
            fetch_block(gut_h, iu, 0, sem_a)
            pltpu.make_async_copy(
                gut_h.at[:, pl.ds(0, 128)], blks.at[0], sem_a).wait()
            extract(iu, 0, ccols, 0)
            pltpu.sync_copy(ici_h.at[pl.ds(0, 16)], cidx)
            ii = scalar_at(cidx, 0)
            fetch_block(git_h, ii, 0, sem_a)
            pltpu.make_async_copy(
                git_h.at[:, pl.ds(0, 128)], blks.at[0], sem_a).wait()
            extract(ii, 0, ccols, 1)
            pltpu.sync_copy(ccols, oc)

    return gk(gut, git, idx_nu, idx_ni, idx_cu, idx_ci)


def _tc_body(xtu_ref, xti_ref, c_ref, wrt_ref, bc_ref, out_ref):
    f32 = jnp.float32
    wrt = wrt_ref[...]
    bc = bc_ref[:, 0:1]
    rr = lax.broadcasted_iota(jnp.int32, (EMBED_K, EMBED_K), 0) // D_K
    cc = lax.broadcasted_iota(jnp.int32, (EMBED_K, EMBED_K), 1) // D_K
    s_mat = (rr == cc).astype(f32)

    def project(xt):
        h = jnp.maximum(
            jnp.dot(wrt, xt, preferred_element_type=f32) + bc, 0.0)
        ss = jnp.dot(s_mat, h * h, preferred_element_type=f32)
        return h / (jnp.sqrt(ss) + EPS)

    zt_u = project(xtu_ref[...])
    zt_i = project(xti_ref[...])
    zc = project(c_ref[...])
    e_u = zc[:, 0:1]
    e_i = zc[:, 1:2]

    def routing(zt, e):
        logits = jnp.dot(s_mat, zt * e, preferred_element_type=f32) * (1.0 / TAU)
        m = jnp.max(logits, axis=0, keepdims=True)
        ex = jnp.exp(logits - m)
        denom = jnp.sum(ex, axis=0, keepdims=True) * (1.0 / D_K)
        p = ex / denom
        agg = jnp.sum(p * zt, axis=1, keepdims=True)
        v = e + agg
        ss = jnp.dot(s_mat, v * v, preferred_element_type=f32)
        return v / (jnp.sqrt(ss) + EPS)

    for _ in range(ROUTING_ITERS):
        e_u = routing(zt_u, e_u)
        e_i = routing(zt_i, e_i)

    xui = jnp.sum(e_u * e_i)
    out_ref[:, 0:1] = e_u
    out_ref[:, 1:2] = e_i
    out_ref[:, 2:3] = jnp.zeros((EMBED_K, 1), f32) + xui
    out_ref[:, 3:8] = jnp.zeros((EMBED_K, 5), f32)


def kernel(Gu, Gi, W, b, user, item, neigh_user, neigh_item):
    gut = Gu.T
    git = Gi.T
    idx_cu = jnp.broadcast_to(user, (16,)).astype(jnp.int32)
    idx_ci = jnp.broadcast_to(item, (16,)).astype(jnp.int32)
    xtu, xti, c2 = _sc_gather(gut, git, neigh_user, neigh_item,
                              idx_cu, idx_ci)
    wrt = jnp.transpose(W, (0, 2, 1)).reshape(DISEN_K * D_K, EMBED_K)
    bc = jnp.broadcast_to(b.reshape(DISEN_K * D_K, 1), (DISEN_K * D_K, 8))
    out = pl.pallas_call(
        _tc_body,
        out_shape=jax.ShapeDtypeStruct((EMBED_K, 8), jnp.float32),
    )(xtu, xti, c2, wrt, bc)
    return (out[0:1, 2].reshape(1), out[:, 0], out[:, 1])
